# baseline (speedup 1.0000x reference)
import math

import jax
import jax.numpy as jnp
from jax.experimental import pallas as pl
from jax.experimental.pallas import tpu as pltpu

_B = 512
_NG = 50
_H = 50
_CAT = 18
_NCAT = 19
_D = 256
_DEPTH = 3
_UG = 68
_SCALAR = math.sqrt(float(_D))
_BB = 8


def _bmm(a, b):
    return jnp.stack(
        [jax.lax.dot(a[i], b[i]) for i in range(a.shape[0])], axis=0
    )


def _gat(x, adj, W, bvec, a1, a2):
    bb, n, d = x.shape
    x2 = x.reshape(bb * n, d)
    h2 = jax.lax.dot(x2, W.T) + bvec
    s1 = jnp.sum(h2 * a1, axis=1).reshape(bb, n)
    s2 = jnp.sum(h2 * a2, axis=1).reshape(bb, n)
    e = s1[:, None, :] + s2[:, :, None]
    e = jnp.where(e >= 0.0, e, 0.2 * e)
    e = jnp.where(adj == 0, -1e9, e)
    m = jnp.max(e, axis=2, keepdims=True)
    ex = jnp.exp(e - m)
    alpha = ex / jnp.sum(ex, axis=2, keepdims=True)
    h = h2.reshape(bb, n, d)
    out = _bmm(alpha, h)
    return jnp.maximum(out, 0.0) + x


def _masked_softmax_pool(feat, query, Kw, Qw, Qb, mask):
    bb, n, d = feat.shape
    k = jax.lax.dot(feat.reshape(bb * n, d), Kw.T).reshape(bb, n, d)
    q = jax.lax.dot(query, Qw.T) + Qb
    a = jnp.sum(k * q[:, None, :], axis=2) / _SCALAR
    a = jnp.where(mask != 0, a, -1e9)
    m = jnp.max(a, axis=1, keepdims=True)
    ex = jnp.exp(a - m)
    alpha = ex / jnp.sum(ex, axis=1, keepdims=True)
    return jnp.sum(alpha[:, :, None] * feat, axis=1)


def _fwd_kernel(nge_ref, une_ref, ngraph_ref, ugraph_ref, ngmask_ref,
                ucmask_ref, ucidx_ref, tne_ref, cand_K_ref, cand_Qw_ref,
                cand_Qb_ref, news_W_w_ref, news_W_b_ref, unK_ref, unQ_ref,
                unQb_ref, feat_w_ref, feat_b_ref, usr_K_ref, usr_Qw_ref,
                usr_Qb_ref, ngat_W_ref, ngat_Wb_ref, ngat_a1_ref,
                ngat_a2_ref, ugat_W_ref, ugat_Wb_ref, ugat_a1_ref,
                ugat_a2_ref, out_ref):
    nge = nge_ref[...]
    une = une_ref[...]
    tne = tne_ref[...]
    ngraph = ngraph_ref[...]
    ugraph = ugraph_ref[...]

    uge = jnp.concatenate(
        [une, jnp.broadcast_to(tne[None], (_BB, _CAT, _D))], axis=1
    )

    for i in range(_DEPTH):
        nge = _gat(nge, ngraph, ngat_W_ref[i], ngat_Wb_ref[i:i + 1, :],
                   ngat_a1_ref[i:i + 1, :], ngat_a2_ref[i:i + 1, :])
        uge = _gat(uge, ugraph, ugat_W_ref[i], ugat_Wb_ref[i:i + 1, :],
                   ugat_a1_ref[i:i + 1, :], ugat_a2_ref[i:i + 1, :])

    local = nge[:, 0, :]
    glob = _masked_softmax_pool(nge, local, cand_K_ref[...], cand_Qw_ref[...],
                                cand_Qb_ref[...], ngmask_ref[...])
    cat = jnp.concatenate([local, glob], axis=1)
    gate = jax.lax.dot(cat, news_W_w_ref[...]) + news_W_b_ref[...]
    gate = 1.0 / (1.0 + jnp.exp(-gate))
    news_ctx = gate * local + (1.0 - gate) * glob

    hist = uge[:, :_H, :]
    kh = jax.lax.dot(hist.reshape(_BB * _H, _D), unK_ref[...].T)
    kh = kh.reshape(_BB, _H, _D)
    qv = jax.lax.dot(news_ctx, unQ_ref[...].T) + unQb_ref[...]
    a = jnp.sum(kh * qv[:, None, :], axis=2) / _SCALAR

    idx = ucidx_ref[...]
    cat_iota = jax.lax.broadcasted_iota(jnp.int32, (_BB, _H, _NCAT), 2)
    onehot = idx[:, :, None] == cat_iota
    mxc = jnp.max(jnp.where(onehot, a[:, :, None], -1e9), axis=1)
    mxl = jnp.sum(jnp.where(onehot, mxc[:, None, :], 0.0), axis=2)
    ex = jnp.exp(a - mxl)
    smc = jnp.sum(jnp.where(onehot, ex[:, :, None], 0.0), axis=1)
    sml = jnp.sum(jnp.where(onehot, smc[:, None, :], 0.0), axis=2)
    alpha = ex / (sml + 1e-16)

    vals = alpha[:, :, None] * hist
    onehot_f = onehot.astype(jnp.float32)
    dnums = (((0,), (0,)), ((), ()))
    topic = jnp.stack(
        [jax.lax.dot_general(onehot_f[i], vals[i], dnums)
         for i in range(_BB)], axis=0)

    t2 = jax.lax.dot(topic.reshape(_BB * _NCAT, _D), feat_w_ref[...].T)
    t2 = t2 + feat_b_ref[...]
    topic2 = (jnp.maximum(t2, 0.0).reshape(_BB, _NCAT, _D) + topic)

    user_ctx = _masked_softmax_pool(topic2, news_ctx, usr_K_ref[...],
                                    usr_Qw_ref[...], usr_Qb_ref[...],
                                    ucmask_ref[...])

    out_ref[...] = jnp.concatenate([news_ctx, user_ctx], axis=1)


def kernel(news_graph_embeddings, user_news_embedding, topic_node_embedding,
           cand_K, cand_Qw, cand_Qb, news_W_w, news_W_b, user_news_K_w,
           user_news_Q_w, user_news_Q_b, feat_w, feat_b, usr_K, usr_Qw,
           usr_Qb, ngat_W, ngat_Wb, ngat_a1, ngat_a2, ugat_W, ugat_Wb,
           ugat_a1, ugat_a2, news_graph, news_graph_mask, user_graph,
           user_category_mask, user_category_indices):
    ngmask = news_graph_mask.astype(jnp.int32)
    ucmask = user_category_mask.astype(jnp.int32)

    def b3(s):
        return pl.BlockSpec(s, lambda i: (i, 0, 0))

    def b2(s):
        return pl.BlockSpec(s, lambda i: (i, 0))

    def f2(s):
        return pl.BlockSpec(s, lambda i: (0, 0))

    def f3(s):
        return pl.BlockSpec(s, lambda i: (0, 0, 0))

    args = (
        news_graph_embeddings,
        user_news_embedding,
        news_graph,
        user_graph,
        ngmask,
        ucmask,
        user_category_indices,
        topic_node_embedding,
        cand_K, cand_Qw, cand_Qb.reshape(1, _D),
        news_W_w.T,
        news_W_b.reshape(1, _D),
        user_news_K_w, user_news_Q_w, user_news_Q_b.reshape(1, _D),
        feat_w, feat_b.reshape(1, _D),
        usr_K, usr_Qw, usr_Qb.reshape(1, _D),
        ngat_W, ngat_Wb, ngat_a1, ngat_a2,
        ugat_W, ugat_Wb, ugat_a1, ugat_a2,
    )
    in_specs = [
        b3((_BB, _NG, _D)),
        b3((_BB, _H, _D)),
        b3((_BB, _NG, _NG)),
        b3((_BB, _UG, _UG)),
        b2((_BB, _NG)),
        b2((_BB, _NCAT)),
        b2((_BB, _H)),
        f2((_CAT, _D)),
        f2((_D, _D)), f2((_D, _D)), f2((1, _D)),
        f2((2 * _D, _D)),
        f2((1, _D)),
        f2((_D, _D)), f2((_D, _D)), f2((1, _D)),
        f2((_D, _D)), f2((1, _D)),
        f2((_D, _D)), f2((_D, _D)), f2((1, _D)),
        f3((_DEPTH, _D, _D)), f2((_DEPTH, _D)), f2((_DEPTH, _D)),
        f2((_DEPTH, _D)),
        f3((_DEPTH, _D, _D)), f2((_DEPTH, _D)), f2((_DEPTH, _D)),
        f2((_DEPTH, _D)),
    ]
    return pl.pallas_call(
        _fwd_kernel,
        grid=(_B // _BB,),
        in_specs=in_specs,
        out_specs=pl.BlockSpec((_BB, 2 * _D), lambda i: (i, 0)),
        out_shape=jax.ShapeDtypeStruct((_B, 2 * _D), jnp.float32),
        compiler_params=pltpu.CompilerParams(
            dimension_semantics=("parallel",)),
    )(*args)
